# baseline (speedup 1.0000x reference)
import functools

import jax
import jax.numpy as jnp
from jax import lax
from jax.experimental import pallas as pl
from jax.experimental.pallas import tpu as pltpu
from jax.experimental.pallas import tpu_sc as plsc


def _fps_body(npoint, coords_ref, idx_ref, newc_ref, dist_ref):
    _, B, N = coords_ref.shape
    BLK = 128
    dist_ref[...] = jnp.full((B, N), 1e10, dtype=jnp.float32)
    lane = lax.broadcasted_iota(jnp.int32, (B, BLK), 1)

    CH = 256
    nch = N // CH
    io0 = lax.broadcasted_iota(jnp.int32, (B, CH), 1)

    def inner(k, carry):
        nf, cx, cy, cz, bi, bx, by, bz = carry
        sel = lane == k
        bi = jnp.where(sel, nf, bi)
        bx = jnp.where(sel, cx, bx)
        by = jnp.where(sel, cy, by)
        bz = jnp.where(sel, cz, bz)
        cur = [None, None]
        curi = [None, None]
        curx = [None, None]
        cury = [None, None]
        curz = [None, None]
        for c in range(nch):
            sl = pl.ds(c * CH, CH)
            xk = coords_ref[0, :, sl]
            yk = coords_ref[1, :, sl]
            zk = coords_ref[2, :, sl]
            dx = xk - cx
            dy = yk - cy
            dz = zk - cz
            d = (dx * dx + dz * dz) + dy * dy
            dn = jnp.minimum(dist_ref[:, sl], d)
            dist_ref[:, sl] = dn
            w = c % 2
            if cur[w] is None:
                cur[w], curi[w] = dn, io0 + c * CH if c else io0
                curx[w], cury[w], curz[w] = xk, yk, zk
            else:
                gt = dn > cur[w]
                cur[w] = jnp.maximum(cur[w], dn)
                curi[w] = jnp.where(gt, io0 + c * CH, curi[w])
                curx[w] = jnp.where(gt, xk, curx[w])
                cury[w] = jnp.where(gt, yk, cury[w])
                curz[w] = jnp.where(gt, zk, curz[w])
        pick = (cur[0] > cur[1]) | ((cur[0] == cur[1]) & (curi[0] < curi[1]))
        curv = jnp.maximum(cur[0], cur[1])
        curiv = jnp.where(pick, curi[0], curi[1])
        curxv = jnp.where(pick, curx[0], curx[1])
        curyv = jnp.where(pick, cury[0], cury[1])
        curzv = jnp.where(pick, curz[0], curz[1])
        m = jnp.max(curv, axis=1, keepdims=True)
        cand = jnp.where(curv == m, curiv, N)
        nf2 = jnp.min(cand, axis=1, keepdims=True)
        oh = curiv == nf2
        cx2 = jnp.sum(jnp.where(oh, curxv, 0.0), axis=1, keepdims=True)
        cy2 = jnp.sum(jnp.where(oh, curyv, 0.0), axis=1, keepdims=True)
        cz2 = jnp.sum(jnp.where(oh, curzv, 0.0), axis=1, keepdims=True)
        return (nf2, cx2, cy2, cz2, bi, bx, by, bz)

    def outer(j, carry):
        nf, cx, cy, cz = carry
        zi = jnp.zeros((B, BLK), jnp.int32)
        zf = jnp.zeros((B, BLK), jnp.float32)
        def inner2(k2, carry):
            carry = inner(2 * k2, carry)
            return inner(2 * k2 + 1, carry)

        nf, cx, cy, cz, bi, bx, by, bz = lax.fori_loop(
            0, BLK // 2, inner2, (nf, cx, cy, cz, zi, zf, zf, zf)
        )
        base = pl.multiple_of(j * BLK, BLK)
        idx_ref[:, pl.ds(base, BLK)] = bi
        newc_ref[0, :, pl.ds(base, BLK)] = bx
        newc_ref[1, :, pl.ds(base, BLK)] = by
        newc_ref[2, :, pl.ds(base, BLK)] = bz
        return (nf, cx, cy, cz)

    lax.fori_loop(
        0,
        npoint // BLK,
        outer,
        (
            jnp.zeros((B, 1), jnp.int32),
            coords_ref[0, :, 0:1],
            coords_ref[1, :, 0:1],
            coords_ref[2, :, 0:1],
        ),
    )


def _fps(coords, npoint, interpret=False):
    C, B, N = coords.shape
    return pl.pallas_call(
        functools.partial(_fps_body, npoint),
        out_shape=(
            jax.ShapeDtypeStruct((B, npoint), jnp.int32),
            jax.ShapeDtypeStruct((C, B, npoint), jnp.float32),
        ),
        scratch_shapes=[pltpu.VMEM((B, N), jnp.float32)],
        interpret=interpret,
    )(coords)


def _values_gather(values_rows, idx, N):
    BN, D = values_rows.shape
    B, S = idx.shape
    info = plsc.get_sparse_core_info()
    nw = info.num_cores * info.num_subcores
    per = nw // B
    spw = S // per
    CH = 128

    @functools.partial(
        pl.kernel,
        mesh=plsc.VectorSubcoreMesh(core_axis_name="c", subcore_axis_name="s"),
        compiler_params=pltpu.CompilerParams(use_tc_tiling_on_sc=False),
        out_type=jax.ShapeDtypeStruct((B * S, D), jnp.float32),
        scratch_types=[
            pltpu.VMEM((spw,), jnp.int32),
            pltpu.VMEM((spw, D), jnp.float32),
            pltpu.SemaphoreType.DMA,
        ],
    )
    def gather_k(table_hbm, idx_hbm, out_hbm, idx_v, rows_v, sem):
        wid = lax.axis_index("s") * info.num_cores + lax.axis_index("c")
        b = wid // per
        s0 = (wid % per) * spw
        pltpu.sync_copy(idx_hbm.at[b, pl.ds(s0, spw)], idx_v)
        off = b * N

        def addoff(k, c):
            idx_v[pl.ds(k * 16, 16)] = idx_v[pl.ds(k * 16, 16)] + off
            return c

        lax.fori_loop(0, spw // 16, addoff, 0)

        copies = [
            pltpu.async_copy(
                table_hbm.at[idx_v.at[pl.ds(c * CH, CH)]],
                rows_v.at[pl.ds(c * CH, CH), :],
                sem,
            )
            for c in range(spw // CH)
        ]
        for cp in copies:
            cp.wait()
        row0 = b * S + s0
        pltpu.sync_copy(rows_v, out_hbm.at[pl.ds(row0, spw), :])

    return gather_k(values_rows, idx)


def kernel(coords, values):
    B, C, N = coords.shape
    D = values.shape[1]
    npoint = N // 2
    fps_idx, newc_cm = _fps(jnp.transpose(coords, (1, 0, 2)), npoint)
    new_coords = jnp.transpose(newc_cm, (1, 0, 2))
    values_rows = jnp.transpose(values, (0, 2, 1)).reshape(B * N, D)
    gathered = _values_gather(values_rows, fps_idx, N)
    new_values = jnp.transpose(gathered.reshape(B, npoint, D), (0, 2, 1))
    return (new_coords, new_values)


# speedup vs baseline: 1.0389x; 1.0389x over previous
import functools

import jax
import jax.numpy as jnp
from jax import lax
from jax.experimental import pallas as pl
from jax.experimental.pallas import tpu as pltpu
from jax.experimental.pallas import tpu_sc as plsc


def _fps_body(npoint, coords_ref, idx_ref, newc_ref, dist_ref):
    _, B, N = coords_ref.shape
    BLK = 128
    dist_ref[...] = jnp.full((B, N), 1e10, dtype=jnp.float32)
    lane = lax.broadcasted_iota(jnp.int32, (B, BLK), 1)

    CH = 512
    nch = N // CH
    io0 = lax.broadcasted_iota(jnp.int32, (B, CH), 1)

    def inner(k, carry):
        far, bi, bx, by, bz = carry
        accx = accy = accz = None
        for c in range(nch):
            sl = pl.ds(c * CH, CH)
            oh = io0 == (far - c * CH)
            px = jnp.where(oh, coords_ref[0, :, sl], 0.0)
            py = jnp.where(oh, coords_ref[1, :, sl], 0.0)
            pz = jnp.where(oh, coords_ref[2, :, sl], 0.0)
            accx = px if c == 0 else accx + px
            accy = py if c == 0 else accy + py
            accz = pz if c == 0 else accz + pz
        cx = jnp.sum(accx, axis=1, keepdims=True)
        cy = jnp.sum(accy, axis=1, keepdims=True)
        cz = jnp.sum(accz, axis=1, keepdims=True)
        sel = lane == k
        bi = jnp.where(sel, far, bi)
        bx = jnp.where(sel, cx, bx)
        by = jnp.where(sel, cy, by)
        bz = jnp.where(sel, cz, bz)
        pm = None
        for c in range(nch):
            sl = pl.ds(c * CH, CH)
            dx = coords_ref[0, :, sl] - cx
            dy = coords_ref[1, :, sl] - cy
            dz = coords_ref[2, :, sl] - cz
            d = (dx * dx + dz * dz) + dy * dy
            dn = jnp.minimum(dist_ref[:, sl], d)
            dist_ref[:, sl] = dn
            pm = dn if c == 0 else jnp.maximum(pm, dn)
        m = jnp.max(pm, axis=1, keepdims=True)
        cm = None
        for c in range(nch):
            sl = pl.ds(c * CH, CH)
            candc = jnp.where(dist_ref[:, sl] == m, io0 + c * CH, N)
            cm = candc if c == 0 else jnp.minimum(cm, candc)
        nf = jnp.min(cm, axis=1, keepdims=True)
        return (nf, bi, bx, by, bz)

    def outer(j, far):
        zi = jnp.zeros((B, BLK), jnp.int32)
        zf = jnp.zeros((B, BLK), jnp.float32)
        far, bi, bx, by, bz = lax.fori_loop(0, BLK, inner, (far, zi, zf, zf, zf))
        base = pl.multiple_of(j * BLK, BLK)
        idx_ref[:, pl.ds(base, BLK)] = bi
        newc_ref[0, :, pl.ds(base, BLK)] = bx
        newc_ref[1, :, pl.ds(base, BLK)] = by
        newc_ref[2, :, pl.ds(base, BLK)] = bz
        return far

    lax.fori_loop(0, npoint // BLK, outer, jnp.zeros((B, 1), jnp.int32))


def _fps(coords, npoint, interpret=False):
    C, B, N = coords.shape
    return pl.pallas_call(
        functools.partial(_fps_body, npoint),
        out_shape=(
            jax.ShapeDtypeStruct((B, npoint), jnp.int32),
            jax.ShapeDtypeStruct((C, B, npoint), jnp.float32),
        ),
        scratch_shapes=[pltpu.VMEM((B, N), jnp.float32)],
        interpret=interpret,
    )(coords)


def _values_gather(values_rows, idx, N):
    BN, D = values_rows.shape
    B, S = idx.shape
    info = plsc.get_sparse_core_info()
    nw = info.num_cores * info.num_subcores
    per = nw // B
    spw = S // per
    CH = 128

    @functools.partial(
        pl.kernel,
        mesh=plsc.VectorSubcoreMesh(core_axis_name="c", subcore_axis_name="s"),
        compiler_params=pltpu.CompilerParams(use_tc_tiling_on_sc=False),
        out_type=jax.ShapeDtypeStruct((B * S, D), jnp.float32),
        scratch_types=[
            pltpu.VMEM((spw,), jnp.int32),
            pltpu.VMEM((spw, D), jnp.float32),
            pltpu.SemaphoreType.DMA,
        ],
    )
    def gather_k(table_hbm, idx_hbm, out_hbm, idx_v, rows_v, sem):
        wid = lax.axis_index("s") * info.num_cores + lax.axis_index("c")
        b = wid // per
        s0 = (wid % per) * spw
        pltpu.sync_copy(idx_hbm.at[b, pl.ds(s0, spw)], idx_v)
        off = b * N

        def addoff(k, c):
            idx_v[pl.ds(k * 16, 16)] = idx_v[pl.ds(k * 16, 16)] + off
            return c

        lax.fori_loop(0, spw // 16, addoff, 0)

        copies = [
            pltpu.async_copy(
                table_hbm.at[idx_v.at[pl.ds(c * CH, CH)]],
                rows_v.at[pl.ds(c * CH, CH), :],
                sem,
            )
            for c in range(spw // CH)
        ]
        for cp in copies:
            cp.wait()
        row0 = b * S + s0
        pltpu.sync_copy(rows_v, out_hbm.at[pl.ds(row0, spw), :])

    return gather_k(values_rows, idx)


def kernel(coords, values):
    B, C, N = coords.shape
    D = values.shape[1]
    npoint = N // 2
    fps_idx, newc_cm = _fps(jnp.transpose(coords, (1, 0, 2)), npoint)
    new_coords = jnp.transpose(newc_cm, (1, 0, 2))
    values_rows = jnp.transpose(values, (0, 2, 1)).reshape(B * N, D)
    gathered = _values_gather(values_rows, fps_idx, N)
    new_values = jnp.transpose(gathered.reshape(B, npoint, D), (0, 2, 1))
    return (new_coords, new_values)
